# baseline (speedup 1.0000x reference)
import functools

import jax
import jax.numpy as jnp
import numpy as np
from jax import lax
from jax.experimental import pallas as pl
from jax.experimental.pallas import tpu as pltpu
from jax.experimental.pallas import tpu_sc as plsc

_EOS = 2
_BEAM = 8
_NEG_INF = np.float32(-np.inf)

_NC = 2
_NS = 16
_NW = _NC * _NS


_CW = 512
_DEPTH = 3


def _ce(v, jv, regs, jregs):
    for d in range(_DEPTH):
        c = v > regs[d]
        regs[d], v = jnp.where(c, v, regs[d]), jnp.where(c, regs[d], v)
        jregs[d], jv = jnp.where(c, jv, jregs[d]), jnp.where(c, jregs[d], jv)
    return regs, jregs


def _stats_topk_body(out_ref, m_ref, s_ref, topv_ref, topi_ref):
    rb = out_ref.shape[0]
    v = out_ref.shape[1]
    nfull = v // _CW
    rem = v - nfull * _CW
    lane = lax.broadcasted_iota(jnp.int32, (rb, _CW), 1)

    ninf = jnp.full((rb, _CW), _NEG_INF)
    zi = jnp.zeros((rb, _CW), jnp.int32)
    init = (ninf, ninf, ninf, zi, zi, zi)

    def s1(j, carry):
        regs, jregs = list(carry[:_DEPTH]), list(carry[_DEPTH:])
        base = j * _CW
        blk = out_ref[:, pl.ds(base, _CW)]
        jb = jnp.full((rb, _CW), base, jnp.int32)
        regs, jregs = _ce(blk, jb, regs, jregs)
        return (*regs, *jregs)

    carry = lax.fori_loop(0, nfull, s1, init)
    regs, jregs = list(carry[:_DEPTH]), list(carry[_DEPTH:])
    if rem:
        blk = out_ref[:, v - _CW : v]
        blk = jnp.where(lane < _CW - rem, _NEG_INF, blk)
        jb = jnp.full((rb, _CW), v - _CW, jnp.int32)
        regs, jregs = _ce(blk, jb, regs, jregs)
    r0 = regs[0]

    m = jnp.max(r0, axis=1, keepdims=True)
    m_ref[...] = m

    rc = r0
    for _ in range(_BEAM - 1):
        vk = jnp.max(rc, axis=1, keepdims=True)
        rc = jnp.where(rc == vk, _NEG_INF, rc)
    t = jnp.max(rc, axis=1, keepdims=True)

    def s2(j, carry):
        acc, cnt = carry
        blk = out_ref[:, pl.ds(j * _CW, _CW)]
        acc = acc + jnp.exp(blk - m)
        cnt = cnt + (blk >= t).astype(jnp.int32)
        return acc, cnt

    acc, cnt = lax.fori_loop(0, nfull, s2, (jnp.zeros((rb, _CW)), zi))
    if rem:
        blk = out_ref[:, v - _CW : v]
        blk = jnp.where(lane < _CW - rem, _NEG_INF, blk)
        acc = acc + jnp.exp(blk - m)
        cnt = cnt + (blk >= t).astype(jnp.int32)
    s_ref[...] = jnp.sum(acc, axis=1, keepdims=True)

    bad = jnp.max(cnt) > _DEPTH

    @pl.when(jnp.logical_not(bad))
    def _fast():
        cv = jnp.concatenate(regs, axis=1)
        li = lax.broadcasted_iota(jnp.int32, (rb, _DEPTH * _CW), 1) & (
            _CW - 1
        )
        ci = jnp.concatenate(jregs, axis=1) + li
        big = jnp.int32(v)
        for k in range(_BEAM):
            vk = jnp.max(cv, axis=1, keepdims=True)
            fi = jnp.min(jnp.where(cv == vk, ci, big), axis=1, keepdims=True)
            topv_ref[:, k : k + 1] = vk
            topi_ref[:, k : k + 1] = fi
            cv = jnp.where((cv == vk) & (ci == fi), _NEG_INF, cv)

    @pl.when(bad)
    def _slow():
        x = out_ref[...]
        iota = lax.broadcasted_iota(jnp.int32, (rb, v), 1)
        big = jnp.int32(v)
        for k in range(_BEAM):
            vk = jnp.max(x, axis=1, keepdims=True)
            fi = jnp.min(jnp.where(x == vk, iota, big), axis=1, keepdims=True)
            topv_ref[:, k : k + 1] = vk
            topi_ref[:, k : k + 1] = fi
            x = jnp.where(iota == fi, _NEG_INF, x)


def _stats_topk(out):
    n, v = out.shape
    rblk = 8
    grid = n // rblk
    return pl.pallas_call(
        _stats_topk_body,
        grid=(grid,),
        in_specs=[pl.BlockSpec((rblk, v), lambda i: (i, 0))],
        out_specs=[
            pl.BlockSpec((rblk, 1), lambda i: (i, 0)),
            pl.BlockSpec((rblk, 1), lambda i: (i, 0)),
            pl.BlockSpec((rblk, _BEAM), lambda i: (i, 0)),
            pl.BlockSpec((rblk, _BEAM), lambda i: (i, 0)),
        ],
        out_shape=[
            jax.ShapeDtypeStruct((n, 1), jnp.float32),
            jax.ShapeDtypeStruct((n, 1), jnp.float32),
            jax.ShapeDtypeStruct((n, _BEAM), jnp.float32),
            jax.ShapeDtypeStruct((n, _BEAM), jnp.int32),
        ],
    )(out)


def _merge_body(v, m_ref, s_ref, sc_ref, tv_ref, ti_ref, ns_ref, x_ref, ptr_ref):
    m = m_ref[...]
    s = s_ref[...]
    sc = sc_ref[...]
    tv = tv_ref[...]
    ti = ti_ref[...]
    b, c = tv.shape
    cand = jnp.exp(tv - m) / s + sc
    iota = lax.broadcasted_iota(jnp.int32, (b, c), 1)
    flat = (iota // _BEAM) * v + ti
    big = jnp.int32(_BEAM * v)
    for k in range(_BEAM):
        vk = jnp.max(cand, axis=1, keepdims=True)
        fk = jnp.min(jnp.where(cand == vk, flat, big), axis=1, keepdims=True)
        ns_ref[:, k : k + 1] = jnp.where(fk % v == _EOS, _NEG_INF, vk)
        x_ref[:, k : k + 1] = fk % v
        ptr_ref[:, k : k + 1] = fk // v
        cand = jnp.where((cand == vk) & (flat == fk), _NEG_INF, cand)


def _merge(m64, s64, sc64, tv64, ti64, v):
    b, c = tv64.shape
    return pl.pallas_call(
        functools.partial(_merge_body, v),
        out_shape=[
            jax.ShapeDtypeStruct((b, _BEAM), jnp.float32),
            jax.ShapeDtypeStruct((b, _BEAM), jnp.int32),
            jax.ShapeDtypeStruct((b, _BEAM), jnp.int32),
        ],
    )(m64, s64, sc64, tv64, ti64)


def _sc_gather(hf, idx):
    rows, d = hf.shape
    bpw = rows // _NW
    mesh = plsc.VectorSubcoreMesh(core_axis_name="c", subcore_axis_name="s")

    @functools.partial(
        pl.kernel,
        mesh=mesh,
        out_type=jax.ShapeDtypeStruct((rows, d), jnp.float32),
        scratch_types=[
            pltpu.VMEM((bpw,), jnp.int32),
            pltpu.VMEM((bpw, d), jnp.float32),
            pltpu.SemaphoreType.DMA,
        ],
    )
    def gather_k(hf_hbm, idx_hbm, out_hbm, idx_v, rows_v, sem):
        wid = lax.axis_index("s") * _NC + lax.axis_index("c")
        base = wid * bpw
        pltpu.sync_copy(idx_hbm.at[pl.ds(base, bpw)], idx_v)
        pltpu.async_copy(hf_hbm.at[idx_v], rows_v, sem).wait()
        pltpu.sync_copy(rows_v, out_hbm.at[pl.ds(base, bpw)])

    return gather_k(hf, idx)


def kernel(out, score, h, beam_size):
    n, v = out.shape
    b = n // _BEAM
    l, _, hd = h.shape

    m, s, topv, topi = _stats_topk(out)

    m64 = jnp.repeat(m.reshape(b, _BEAM), _BEAM, axis=1)
    s64 = jnp.repeat(s.reshape(b, _BEAM), _BEAM, axis=1)
    sc64 = jnp.repeat(score.reshape(b, _BEAM), _BEAM, axis=1)
    tv64 = topv.reshape(b, _BEAM * _BEAM)
    ti64 = topi.reshape(b, _BEAM * _BEAM)

    new_score, xt, beam = _merge(m64, s64, sc64, tv64, ti64, v)

    batch_base = jnp.arange(b, dtype=jnp.int32) * _BEAM
    ptr = jnp.reshape(beam + batch_base[:, None], (-1,))
    x = jnp.reshape(xt, (-1,))

    hf = h.reshape(l * n, hd)
    layer_off = jnp.repeat(jnp.arange(l, dtype=jnp.int32) * n, n)
    flat_idx = jnp.tile(ptr, l) + layer_off
    h_new = _sc_gather(hf, flat_idx).reshape(l, n, hd)

    return new_score, x, ptr, h_new


# speedup vs baseline: 1.2199x; 1.2199x over previous
import functools

import jax
import jax.numpy as jnp
import numpy as np
from jax import lax
from jax.experimental import pallas as pl
from jax.experimental.pallas import tpu as pltpu
from jax.experimental.pallas import tpu_sc as plsc

_EOS = 2
_BEAM = 8
_NEG_INF = np.float32(-np.inf)

_NC = 2
_NS = 16
_NW = _NC * _NS


_CW = 512
_DEPTH = 3


def _ce(v, jv, regs, jregs):
    for d in range(_DEPTH):
        c = v > regs[d]
        regs[d], v = jnp.where(c, v, regs[d]), jnp.where(c, regs[d], v)
        jregs[d], jv = jnp.where(c, jv, jregs[d]), jnp.where(c, jregs[d], jv)
    return regs, jregs


def _stats_topk_body(out_ref, m_ref, s_ref, topv_ref, topi_ref):
    rb = out_ref.shape[0]
    v = out_ref.shape[1]
    nfull = v // _CW
    rem = v - nfull * _CW
    lane = lax.broadcasted_iota(jnp.int32, (rb, _CW), 1)

    ninf = jnp.full((rb, _CW), _NEG_INF)
    zi = jnp.zeros((rb, _CW), jnp.int32)
    init = (ninf, ninf, ninf, zi, zi, zi)

    def s1(j, carry):
        regs, jregs = list(carry[:_DEPTH]), list(carry[_DEPTH:])
        base = j * _CW
        blk = out_ref[:, pl.ds(base, _CW)]
        jb = jnp.full((rb, _CW), base, jnp.int32)
        regs, jregs = _ce(blk, jb, regs, jregs)
        return (*regs, *jregs)

    carry = lax.fori_loop(0, nfull, s1, init)
    regs, jregs = list(carry[:_DEPTH]), list(carry[_DEPTH:])
    if rem:
        blk = out_ref[:, v - _CW : v]
        blk = jnp.where(lane < _CW - rem, _NEG_INF, blk)
        jb = jnp.full((rb, _CW), v - _CW, jnp.int32)
        regs, jregs = _ce(blk, jb, regs, jregs)
    r0 = regs[0]

    m = jnp.max(r0, axis=1, keepdims=True)
    m_ref[...] = m

    rc = r0
    for _ in range(_BEAM - 1):
        vk = jnp.max(rc, axis=1, keepdims=True)
        rc = jnp.where(rc == vk, _NEG_INF, rc)
    t = jnp.max(rc, axis=1, keepdims=True)

    def s2(j, carry):
        acc, cnt = carry
        blk = out_ref[:, pl.ds(j * _CW, _CW)]
        acc = acc + jnp.exp(blk - m)
        cnt = cnt + (blk >= t).astype(jnp.int32)
        return acc, cnt

    acc, cnt = lax.fori_loop(0, nfull, s2, (jnp.zeros((rb, _CW)), zi))
    if rem:
        blk = out_ref[:, v - _CW : v]
        blk = jnp.where(lane < _CW - rem, _NEG_INF, blk)
        acc = acc + jnp.exp(blk - m)
        cnt = cnt + (blk >= t).astype(jnp.int32)
    s_ref[...] = jnp.sum(acc, axis=1, keepdims=True)

    bad = jnp.max(cnt) > _DEPTH

    @pl.when(jnp.logical_not(bad))
    def _fast():
        cv = jnp.concatenate(regs, axis=1)
        li = lax.broadcasted_iota(jnp.int32, (rb, _DEPTH * _CW), 1) & (
            _CW - 1
        )
        ci = jnp.concatenate(jregs, axis=1) + li
        big = jnp.int32(v)
        for k in range(_BEAM):
            vk = jnp.max(cv, axis=1, keepdims=True)
            fi = jnp.min(jnp.where(cv == vk, ci, big), axis=1, keepdims=True)
            topv_ref[:, k : k + 1] = vk
            topi_ref[:, k : k + 1] = fi
            cv = jnp.where((cv == vk) & (ci == fi), _NEG_INF, cv)

    @pl.when(bad)
    def _slow():
        x = out_ref[...]
        iota = lax.broadcasted_iota(jnp.int32, (rb, v), 1)
        big = jnp.int32(v)
        for k in range(_BEAM):
            vk = jnp.max(x, axis=1, keepdims=True)
            fi = jnp.min(jnp.where(x == vk, iota, big), axis=1, keepdims=True)
            topv_ref[:, k : k + 1] = vk
            topi_ref[:, k : k + 1] = fi
            x = jnp.where(iota == fi, _NEG_INF, x)


def _stats_topk(out):
    n, v = out.shape
    rblk = 16
    grid = n // rblk
    return pl.pallas_call(
        _stats_topk_body,
        grid=(grid,),
        in_specs=[pl.BlockSpec((rblk, v), lambda i: (i, 0))],
        out_specs=[
            pl.BlockSpec((rblk, 1), lambda i: (i, 0)),
            pl.BlockSpec((rblk, 1), lambda i: (i, 0)),
            pl.BlockSpec((rblk, _BEAM), lambda i: (i, 0)),
            pl.BlockSpec((rblk, _BEAM), lambda i: (i, 0)),
        ],
        out_shape=[
            jax.ShapeDtypeStruct((n, 1), jnp.float32),
            jax.ShapeDtypeStruct((n, 1), jnp.float32),
            jax.ShapeDtypeStruct((n, _BEAM), jnp.float32),
            jax.ShapeDtypeStruct((n, _BEAM), jnp.int32),
        ],
    )(out)


def _merge_body(v, m_ref, s_ref, sc_ref, tv_ref, ti_ref, ns_ref, x_ref, ptr_ref):
    m = m_ref[...]
    s = s_ref[...]
    sc = sc_ref[...]
    tv = tv_ref[...]
    ti = ti_ref[...]
    b, c = tv.shape
    cand = jnp.exp(tv - m) / s + sc
    iota = lax.broadcasted_iota(jnp.int32, (b, c), 1)
    flat = (iota // _BEAM) * v + ti
    big = jnp.int32(_BEAM * v)
    for k in range(_BEAM):
        vk = jnp.max(cand, axis=1, keepdims=True)
        fk = jnp.min(jnp.where(cand == vk, flat, big), axis=1, keepdims=True)
        ns_ref[:, k : k + 1] = jnp.where(fk % v == _EOS, _NEG_INF, vk)
        x_ref[:, k : k + 1] = fk % v
        ptr_ref[:, k : k + 1] = fk // v
        cand = jnp.where((cand == vk) & (flat == fk), _NEG_INF, cand)


def _merge(m64, s64, sc64, tv64, ti64, v):
    b, c = tv64.shape
    return pl.pallas_call(
        functools.partial(_merge_body, v),
        out_shape=[
            jax.ShapeDtypeStruct((b, _BEAM), jnp.float32),
            jax.ShapeDtypeStruct((b, _BEAM), jnp.int32),
            jax.ShapeDtypeStruct((b, _BEAM), jnp.int32),
        ],
    )(m64, s64, sc64, tv64, ti64)


def _sc_gather(hf, idx):
    rows, d = hf.shape
    bpw = rows // _NW
    mesh = plsc.VectorSubcoreMesh(core_axis_name="c", subcore_axis_name="s")

    @functools.partial(
        pl.kernel,
        mesh=mesh,
        out_type=jax.ShapeDtypeStruct((rows, d), jnp.float32),
        scratch_types=[
            pltpu.VMEM((bpw,), jnp.int32),
            pltpu.VMEM((bpw, d), jnp.float32),
            pltpu.SemaphoreType.DMA,
        ],
    )
    def gather_k(hf_hbm, idx_hbm, out_hbm, idx_v, rows_v, sem):
        wid = lax.axis_index("s") * _NC + lax.axis_index("c")
        base = wid * bpw
        pltpu.sync_copy(idx_hbm.at[pl.ds(base, bpw)], idx_v)
        pltpu.async_copy(hf_hbm.at[idx_v], rows_v, sem).wait()
        pltpu.sync_copy(rows_v, out_hbm.at[pl.ds(base, bpw)])

    return gather_k(hf, idx)


def kernel(out, score, h, beam_size):
    n, v = out.shape
    b = n // _BEAM
    l, _, hd = h.shape

    m, s, topv, topi = _stats_topk(out)

    m64 = jnp.repeat(m.reshape(b, _BEAM), _BEAM, axis=1)
    s64 = jnp.repeat(s.reshape(b, _BEAM), _BEAM, axis=1)
    sc64 = jnp.repeat(score.reshape(b, _BEAM), _BEAM, axis=1)
    tv64 = topv.reshape(b, _BEAM * _BEAM)
    ti64 = topi.reshape(b, _BEAM * _BEAM)

    new_score, xt, beam = _merge(m64, s64, sc64, tv64, ti64, v)

    batch_base = jnp.arange(b, dtype=jnp.int32) * _BEAM
    ptr = jnp.reshape(beam + batch_base[:, None], (-1,))
    x = jnp.reshape(xt, (-1,))

    hf = h.reshape(l * n, hd)
    layer_off = jnp.repeat(jnp.arange(l, dtype=jnp.int32) * n, n)
    flat_idx = jnp.tile(ptr, l) + layer_off
    h_new = _sc_gather(hf, flat_idx).reshape(l, n, hd)

    return new_score, x, ptr, h_new


# speedup vs baseline: 1.2556x; 1.0293x over previous
import functools

import jax
import jax.numpy as jnp
import numpy as np
from jax import lax
from jax.experimental import pallas as pl
from jax.experimental.pallas import tpu as pltpu
from jax.experimental.pallas import tpu_sc as plsc

_EOS = 2
_BEAM = 8
_NEG_INF = np.float32(-np.inf)

_NC = 2
_NS = 16
_NW = _NC * _NS


_CW = 512
_DEPTH = 3


def _ce(v, jv, regs, jregs):
    for d in range(_DEPTH):
        c = v > regs[d]
        regs[d], v = jnp.where(c, v, regs[d]), jnp.where(c, regs[d], v)
        jregs[d], jv = jnp.where(c, jv, jregs[d]), jnp.where(c, jregs[d], jv)
    return regs, jregs


def _stats_topk_body(out_ref, m_ref, s_ref, topv_ref, topi_ref):
    rb = out_ref.shape[0]
    v = out_ref.shape[1]
    nfull = v // _CW
    rem = v - nfull * _CW
    lane = lax.broadcasted_iota(jnp.int32, (rb, _CW), 1)

    ninf = jnp.full((rb, _CW), _NEG_INF)
    zi = jnp.zeros((rb, _CW), jnp.int32)
    init = (ninf, ninf, ninf, zi, zi, zi)

    def s1(j, carry):
        regs, jregs = list(carry[:_DEPTH]), list(carry[_DEPTH:])
        base = j * _CW
        blk = out_ref[:, pl.ds(base, _CW)]
        jb = jnp.full((rb, _CW), base, jnp.int32)
        regs, jregs = _ce(blk, jb, regs, jregs)
        return (*regs, *jregs)

    carry = lax.fori_loop(0, nfull, s1, init)
    regs, jregs = list(carry[:_DEPTH]), list(carry[_DEPTH:])
    if rem:
        blk = out_ref[:, v - _CW : v]
        blk = jnp.where(lane < _CW - rem, _NEG_INF, blk)
        jb = jnp.full((rb, _CW), v - _CW, jnp.int32)
        regs, jregs = _ce(blk, jb, regs, jregs)
    r0 = regs[0]

    m = jnp.max(r0, axis=1, keepdims=True)
    m_ref[...] = m

    rc = r0
    for _ in range(_BEAM - 1):
        vk = jnp.max(rc, axis=1, keepdims=True)
        rc = jnp.where(rc == vk, _NEG_INF, rc)
    t = jnp.max(rc, axis=1, keepdims=True)

    def s2(j, carry):
        acc, cnt = carry
        blk = out_ref[:, pl.ds(j * _CW, _CW)]
        acc = acc + jnp.exp(blk - m)
        cnt = cnt + (blk >= t).astype(jnp.int32)
        return acc, cnt

    acc, cnt = lax.fori_loop(0, nfull, s2, (jnp.zeros((rb, _CW)), zi))
    if rem:
        blk = out_ref[:, v - _CW : v]
        blk = jnp.where(lane < _CW - rem, _NEG_INF, blk)
        acc = acc + jnp.exp(blk - m)
        cnt = cnt + (blk >= t).astype(jnp.int32)
    s_ref[...] = jnp.sum(acc, axis=1, keepdims=True)

    bad = jnp.max(cnt) > _DEPTH

    @pl.when(jnp.logical_not(bad))
    def _fast():
        cv = jnp.concatenate(regs, axis=1)
        li = lax.broadcasted_iota(jnp.int32, (rb, _DEPTH * _CW), 1) & (
            _CW - 1
        )
        ci = jnp.concatenate(jregs, axis=1) + li
        big = jnp.int32(v)
        for k in range(_BEAM):
            vk = jnp.max(cv, axis=1, keepdims=True)
            fi = jnp.min(jnp.where(cv == vk, ci, big), axis=1, keepdims=True)
            topv_ref[:, k : k + 1] = vk
            topi_ref[:, k : k + 1] = fi
            cv = jnp.where((cv == vk) & (ci == fi), _NEG_INF, cv)

    @pl.when(bad)
    def _slow():
        x = out_ref[...]
        iota = lax.broadcasted_iota(jnp.int32, (rb, v), 1)
        big = jnp.int32(v)
        for k in range(_BEAM):
            vk = jnp.max(x, axis=1, keepdims=True)
            fi = jnp.min(jnp.where(x == vk, iota, big), axis=1, keepdims=True)
            topv_ref[:, k : k + 1] = vk
            topi_ref[:, k : k + 1] = fi
            x = jnp.where(iota == fi, _NEG_INF, x)


def _stats_topk(out):
    n, v = out.shape
    rblk = 32
    grid = n // rblk
    return pl.pallas_call(
        _stats_topk_body,
        grid=(grid,),
        in_specs=[pl.BlockSpec((rblk, v), lambda i: (i, 0))],
        out_specs=[
            pl.BlockSpec((rblk, 1), lambda i: (i, 0)),
            pl.BlockSpec((rblk, 1), lambda i: (i, 0)),
            pl.BlockSpec((rblk, _BEAM), lambda i: (i, 0)),
            pl.BlockSpec((rblk, _BEAM), lambda i: (i, 0)),
        ],
        out_shape=[
            jax.ShapeDtypeStruct((n, 1), jnp.float32),
            jax.ShapeDtypeStruct((n, 1), jnp.float32),
            jax.ShapeDtypeStruct((n, _BEAM), jnp.float32),
            jax.ShapeDtypeStruct((n, _BEAM), jnp.int32),
        ],
    )(out)


def _merge_body(v, m_ref, s_ref, sc_ref, tv_ref, ti_ref, ns_ref, x_ref, ptr_ref):
    m = m_ref[...]
    s = s_ref[...]
    sc = sc_ref[...]
    tv = tv_ref[...]
    ti = ti_ref[...]
    b, c = tv.shape
    cand = jnp.exp(tv - m) / s + sc
    iota = lax.broadcasted_iota(jnp.int32, (b, c), 1)
    flat = (iota // _BEAM) * v + ti
    big = jnp.int32(_BEAM * v)
    for k in range(_BEAM):
        vk = jnp.max(cand, axis=1, keepdims=True)
        fk = jnp.min(jnp.where(cand == vk, flat, big), axis=1, keepdims=True)
        ns_ref[:, k : k + 1] = jnp.where(fk % v == _EOS, _NEG_INF, vk)
        x_ref[:, k : k + 1] = fk % v
        ptr_ref[:, k : k + 1] = fk // v
        cand = jnp.where((cand == vk) & (flat == fk), _NEG_INF, cand)


def _merge(m64, s64, sc64, tv64, ti64, v):
    b, c = tv64.shape
    return pl.pallas_call(
        functools.partial(_merge_body, v),
        out_shape=[
            jax.ShapeDtypeStruct((b, _BEAM), jnp.float32),
            jax.ShapeDtypeStruct((b, _BEAM), jnp.int32),
            jax.ShapeDtypeStruct((b, _BEAM), jnp.int32),
        ],
    )(m64, s64, sc64, tv64, ti64)


def _sc_gather(hf, idx):
    rows, d = hf.shape
    bpw = rows // _NW
    mesh = plsc.VectorSubcoreMesh(core_axis_name="c", subcore_axis_name="s")

    @functools.partial(
        pl.kernel,
        mesh=mesh,
        out_type=jax.ShapeDtypeStruct((rows, d), jnp.float32),
        scratch_types=[
            pltpu.VMEM((bpw,), jnp.int32),
            pltpu.VMEM((bpw, d), jnp.float32),
            pltpu.SemaphoreType.DMA,
        ],
    )
    def gather_k(hf_hbm, idx_hbm, out_hbm, idx_v, rows_v, sem):
        wid = lax.axis_index("s") * _NC + lax.axis_index("c")
        base = wid * bpw
        pltpu.sync_copy(idx_hbm.at[pl.ds(base, bpw)], idx_v)
        pltpu.async_copy(hf_hbm.at[idx_v], rows_v, sem).wait()
        pltpu.sync_copy(rows_v, out_hbm.at[pl.ds(base, bpw)])

    return gather_k(hf, idx)


def kernel(out, score, h, beam_size):
    n, v = out.shape
    b = n // _BEAM
    l, _, hd = h.shape

    m, s, topv, topi = _stats_topk(out)

    m64 = jnp.repeat(m.reshape(b, _BEAM), _BEAM, axis=1)
    s64 = jnp.repeat(s.reshape(b, _BEAM), _BEAM, axis=1)
    sc64 = jnp.repeat(score.reshape(b, _BEAM), _BEAM, axis=1)
    tv64 = topv.reshape(b, _BEAM * _BEAM)
    ti64 = topi.reshape(b, _BEAM * _BEAM)

    new_score, xt, beam = _merge(m64, s64, sc64, tv64, ti64, v)

    batch_base = jnp.arange(b, dtype=jnp.int32) * _BEAM
    ptr = jnp.reshape(beam + batch_base[:, None], (-1,))
    x = jnp.reshape(xt, (-1,))

    hf = h.reshape(l * n, hd)
    layer_off = jnp.repeat(jnp.arange(l, dtype=jnp.int32) * n, n)
    flat_idx = jnp.tile(ptr, l) + layer_off
    h_new = _sc_gather(hf, flat_idx).reshape(l, n, hd)

    return new_score, x, ptr, h_new


# speedup vs baseline: 1.4019x; 1.1165x over previous
import functools

import jax
import jax.numpy as jnp
import numpy as np
from jax import lax
from jax.experimental import pallas as pl
from jax.experimental.pallas import tpu as pltpu
from jax.experimental.pallas import tpu_sc as plsc

_EOS = 2
_BEAM = 8
_NEG_INF = np.float32(-np.inf)

_NC = 2
_NS = 16
_NW = _NC * _NS


_CW = 512
_DEPTH = 3


def _ce(v, jv, regs, jregs):
    for d in range(_DEPTH - 1):
        c = v > regs[d]
        regs[d], v = jnp.where(c, v, regs[d]), jnp.where(c, regs[d], v)
        jregs[d], jv = jnp.where(c, jv, jregs[d]), jnp.where(c, jregs[d], jv)
    c = v > regs[_DEPTH - 1]
    regs[_DEPTH - 1] = jnp.where(c, v, regs[_DEPTH - 1])
    jregs[_DEPTH - 1] = jnp.where(c, jv, jregs[_DEPTH - 1])
    return regs, jregs


def _stats_topk_body(out_ref, m_ref, s_ref, topv_ref, topi_ref):
    rb = out_ref.shape[0]
    v = out_ref.shape[1]
    nfull = v // _CW
    rem = v - nfull * _CW
    lane = lax.broadcasted_iota(jnp.int32, (rb, _CW), 1)

    ninf = jnp.full((rb, _CW), _NEG_INF)
    zi = jnp.zeros((rb, _CW), jnp.int32)
    init = (ninf, ninf, ninf, zi, zi, zi)

    def s1(j, carry):
        regs, jregs = list(carry[:_DEPTH]), list(carry[_DEPTH:])
        base = j * _CW
        blk = out_ref[:, pl.ds(base, _CW)]
        jb = jnp.full((rb, _CW), base, jnp.int32)
        regs, jregs = _ce(blk, jb, regs, jregs)
        return (*regs, *jregs)

    carry = lax.fori_loop(0, nfull, s1, init, unroll=2)
    regs, jregs = list(carry[:_DEPTH]), list(carry[_DEPTH:])
    if rem:
        blk = out_ref[:, v - _CW : v]
        blk = jnp.where(lane < _CW - rem, _NEG_INF, blk)
        jb = jnp.full((rb, _CW), v - _CW, jnp.int32)
        regs, jregs = _ce(blk, jb, regs, jregs)
    r0 = regs[0]

    m = jnp.max(r0, axis=1, keepdims=True)
    m_ref[...] = m

    rc = r0
    for _ in range(_BEAM - 1):
        vk = jnp.max(rc, axis=1, keepdims=True)
        rc = jnp.where(rc == vk, _NEG_INF, rc)
    t = jnp.max(rc, axis=1, keepdims=True)

    def s2(j, carry):
        acc, cnt = carry
        blk = out_ref[:, pl.ds(j * _CW, _CW)]
        acc = acc + jnp.exp(blk - m)
        cnt = cnt + (blk >= t).astype(jnp.int32)
        return acc, cnt

    acc, cnt = lax.fori_loop(0, nfull, s2, (jnp.zeros((rb, _CW)), zi), unroll=2)
    if rem:
        blk = out_ref[:, v - _CW : v]
        blk = jnp.where(lane < _CW - rem, _NEG_INF, blk)
        acc = acc + jnp.exp(blk - m)
        cnt = cnt + (blk >= t).astype(jnp.int32)
    s_ref[...] = jnp.sum(acc, axis=1, keepdims=True)

    bad = jnp.max(cnt) > _DEPTH

    @pl.when(jnp.logical_not(bad))
    def _fast():
        cv = jnp.concatenate(regs, axis=1)
        li = lax.broadcasted_iota(jnp.int32, (rb, _DEPTH * _CW), 1) & (
            _CW - 1
        )
        ci = jnp.concatenate(jregs, axis=1) + li
        big = jnp.int32(v)
        for k in range(_BEAM):
            vk = jnp.max(cv, axis=1, keepdims=True)
            fi = jnp.min(jnp.where(cv == vk, ci, big), axis=1, keepdims=True)
            topv_ref[:, k : k + 1] = vk
            topi_ref[:, k : k + 1] = fi
            cv = jnp.where((cv == vk) & (ci == fi), _NEG_INF, cv)

    @pl.when(bad)
    def _slow():
        x = out_ref[...]
        iota = lax.broadcasted_iota(jnp.int32, (rb, v), 1)
        big = jnp.int32(v)
        for k in range(_BEAM):
            vk = jnp.max(x, axis=1, keepdims=True)
            fi = jnp.min(jnp.where(x == vk, iota, big), axis=1, keepdims=True)
            topv_ref[:, k : k + 1] = vk
            topi_ref[:, k : k + 1] = fi
            x = jnp.where(iota == fi, _NEG_INF, x)


def _stats_topk(out):
    n, v = out.shape
    rblk = 32
    grid = n // rblk
    return pl.pallas_call(
        _stats_topk_body,
        grid=(grid,),
        in_specs=[pl.BlockSpec((rblk, v), lambda i: (i, 0))],
        out_specs=[
            pl.BlockSpec((rblk, 1), lambda i: (i, 0)),
            pl.BlockSpec((rblk, 1), lambda i: (i, 0)),
            pl.BlockSpec((rblk, _BEAM), lambda i: (i, 0)),
            pl.BlockSpec((rblk, _BEAM), lambda i: (i, 0)),
        ],
        out_shape=[
            jax.ShapeDtypeStruct((n, 1), jnp.float32),
            jax.ShapeDtypeStruct((n, 1), jnp.float32),
            jax.ShapeDtypeStruct((n, _BEAM), jnp.float32),
            jax.ShapeDtypeStruct((n, _BEAM), jnp.int32),
        ],
    )(out)


def _merge_body(v, m_ref, s_ref, sc_ref, tv_ref, ti_ref, ns_ref, x_ref, ptr_ref):
    m = m_ref[...]
    s = s_ref[...]
    sc = sc_ref[...]
    tv = tv_ref[...]
    ti = ti_ref[...]
    b, c = tv.shape
    cand = jnp.exp(tv - m) / s + sc
    iota = lax.broadcasted_iota(jnp.int32, (b, c), 1)
    flat = (iota // _BEAM) * v + ti
    big = jnp.int32(_BEAM * v)
    for k in range(_BEAM):
        vk = jnp.max(cand, axis=1, keepdims=True)
        fk = jnp.min(jnp.where(cand == vk, flat, big), axis=1, keepdims=True)
        ns_ref[:, k : k + 1] = jnp.where(fk % v == _EOS, _NEG_INF, vk)
        x_ref[:, k : k + 1] = fk % v
        ptr_ref[:, k : k + 1] = fk // v
        cand = jnp.where((cand == vk) & (flat == fk), _NEG_INF, cand)


def _merge(m64, s64, sc64, tv64, ti64, v):
    b, c = tv64.shape
    return pl.pallas_call(
        functools.partial(_merge_body, v),
        out_shape=[
            jax.ShapeDtypeStruct((b, _BEAM), jnp.float32),
            jax.ShapeDtypeStruct((b, _BEAM), jnp.int32),
            jax.ShapeDtypeStruct((b, _BEAM), jnp.int32),
        ],
    )(m64, s64, sc64, tv64, ti64)


def _sc_gather(hf, idx):
    rows, d = hf.shape
    bpw = rows // _NW
    mesh = plsc.VectorSubcoreMesh(core_axis_name="c", subcore_axis_name="s")

    @functools.partial(
        pl.kernel,
        mesh=mesh,
        out_type=jax.ShapeDtypeStruct((rows, d), jnp.float32),
        scratch_types=[
            pltpu.VMEM((bpw,), jnp.int32),
            pltpu.VMEM((bpw, d), jnp.float32),
            pltpu.SemaphoreType.DMA,
        ],
    )
    def gather_k(hf_hbm, idx_hbm, out_hbm, idx_v, rows_v, sem):
        wid = lax.axis_index("s") * _NC + lax.axis_index("c")
        base = wid * bpw
        pltpu.sync_copy(idx_hbm.at[pl.ds(base, bpw)], idx_v)
        pltpu.async_copy(hf_hbm.at[idx_v], rows_v, sem).wait()
        pltpu.sync_copy(rows_v, out_hbm.at[pl.ds(base, bpw)])

    return gather_k(hf, idx)


def kernel(out, score, h, beam_size):
    n, v = out.shape
    b = n // _BEAM
    l, _, hd = h.shape

    m, s, topv, topi = _stats_topk(out)

    m64 = jnp.repeat(m.reshape(b, _BEAM), _BEAM, axis=1)
    s64 = jnp.repeat(s.reshape(b, _BEAM), _BEAM, axis=1)
    sc64 = jnp.repeat(score.reshape(b, _BEAM), _BEAM, axis=1)
    tv64 = topv.reshape(b, _BEAM * _BEAM)
    ti64 = topi.reshape(b, _BEAM * _BEAM)

    new_score, xt, beam = _merge(m64, s64, sc64, tv64, ti64, v)

    batch_base = jnp.arange(b, dtype=jnp.int32) * _BEAM
    ptr = jnp.reshape(beam + batch_base[:, None], (-1,))
    x = jnp.reshape(xt, (-1,))

    hf = h.reshape(l * n, hd)
    layer_off = jnp.repeat(jnp.arange(l, dtype=jnp.int32) * n, n)
    flat_idx = jnp.tile(ptr, l) + layer_off
    h_new = _sc_gather(hf, flat_idx).reshape(l, n, hd)

    return new_score, x, ptr, h_new


# speedup vs baseline: 1.4960x; 1.0671x over previous
import functools

import jax
import jax.numpy as jnp
import numpy as np
from jax import lax
from jax.experimental import pallas as pl
from jax.experimental.pallas import tpu as pltpu
from jax.experimental.pallas import tpu_sc as plsc

_EOS = 2
_BEAM = 8
_NEG_INF = np.float32(-np.inf)

_NC = 2
_NS = 16
_NW = _NC * _NS


_CW = 512
_DEPTH = 3


def _ce(v, jv, regs, jregs):
    for d in range(_DEPTH - 1):
        c = v > regs[d]
        regs[d], v = jnp.where(c, v, regs[d]), jnp.where(c, regs[d], v)
        jregs[d], jv = jnp.where(c, jv, jregs[d]), jnp.where(c, jregs[d], jv)
    c = v > regs[_DEPTH - 1]
    regs[_DEPTH - 1] = jnp.where(c, v, regs[_DEPTH - 1])
    jregs[_DEPTH - 1] = jnp.where(c, jv, jregs[_DEPTH - 1])
    return regs, jregs


def _stats_topk_body(out_ref, m_ref, s_ref, topv_ref, topi_ref):
    rb = out_ref.shape[0]
    v = out_ref.shape[1]
    nfull = v // _CW
    rem = v - nfull * _CW
    lane = lax.broadcasted_iota(jnp.int32, (rb, _CW), 1)

    ninf = jnp.full((rb, _CW), _NEG_INF)
    zi = jnp.zeros((rb, _CW), jnp.int32)
    init = (ninf, ninf, ninf, zi, zi, zi)

    def s1(j, carry):
        regs, jregs = list(carry[:_DEPTH]), list(carry[_DEPTH:])
        base = j * _CW
        blk = out_ref[:, pl.ds(base, _CW)]
        jb = jnp.full((rb, _CW), base, jnp.int32)
        regs, jregs = _ce(blk, jb, regs, jregs)
        return (*regs, *jregs)

    carry = lax.fori_loop(0, nfull, s1, init, unroll=4)
    regs, jregs = list(carry[:_DEPTH]), list(carry[_DEPTH:])
    if rem:
        blk = out_ref[:, v - _CW : v]
        blk = jnp.where(lane < _CW - rem, _NEG_INF, blk)
        jb = jnp.full((rb, _CW), v - _CW, jnp.int32)
        regs, jregs = _ce(blk, jb, regs, jregs)
    r0 = regs[0]

    m = jnp.max(r0, axis=1, keepdims=True)
    m_ref[...] = m

    rc = r0
    for _ in range(_BEAM - 1):
        vk = jnp.max(rc, axis=1, keepdims=True)
        rc = jnp.where(rc == vk, _NEG_INF, rc)
    t = jnp.max(rc, axis=1, keepdims=True)

    def s2(j, carry):
        acc, cnt = carry
        blk = out_ref[:, pl.ds(j * _CW, _CW)]
        acc = acc + jnp.exp(blk - m)
        cnt = cnt + (blk >= t).astype(jnp.int32)
        return acc, cnt

    acc, cnt = lax.fori_loop(0, nfull, s2, (jnp.zeros((rb, _CW)), zi), unroll=4)
    if rem:
        blk = out_ref[:, v - _CW : v]
        blk = jnp.where(lane < _CW - rem, _NEG_INF, blk)
        acc = acc + jnp.exp(blk - m)
        cnt = cnt + (blk >= t).astype(jnp.int32)
    s_ref[...] = jnp.sum(acc, axis=1, keepdims=True)

    bad = jnp.max(cnt) > _DEPTH

    @pl.when(jnp.logical_not(bad))
    def _fast():
        cv = jnp.concatenate(regs, axis=1)
        li = lax.broadcasted_iota(jnp.int32, (rb, _DEPTH * _CW), 1) & (
            _CW - 1
        )
        ci = jnp.concatenate(jregs, axis=1) + li
        big = jnp.int32(v)
        for k in range(_BEAM):
            vk = jnp.max(cv, axis=1, keepdims=True)
            fi = jnp.min(jnp.where(cv == vk, ci, big), axis=1, keepdims=True)
            topv_ref[:, k : k + 1] = vk
            topi_ref[:, k : k + 1] = fi
            cv = jnp.where((cv == vk) & (ci == fi), _NEG_INF, cv)

    @pl.when(bad)
    def _slow():
        x = out_ref[...]
        iota = lax.broadcasted_iota(jnp.int32, (rb, v), 1)
        big = jnp.int32(v)
        for k in range(_BEAM):
            vk = jnp.max(x, axis=1, keepdims=True)
            fi = jnp.min(jnp.where(x == vk, iota, big), axis=1, keepdims=True)
            topv_ref[:, k : k + 1] = vk
            topi_ref[:, k : k + 1] = fi
            x = jnp.where(iota == fi, _NEG_INF, x)


def _stats_topk(out):
    n, v = out.shape
    rblk = 32
    grid = n // rblk
    return pl.pallas_call(
        _stats_topk_body,
        grid=(grid,),
        in_specs=[pl.BlockSpec((rblk, v), lambda i: (i, 0))],
        out_specs=[
            pl.BlockSpec((rblk, 1), lambda i: (i, 0)),
            pl.BlockSpec((rblk, 1), lambda i: (i, 0)),
            pl.BlockSpec((rblk, _BEAM), lambda i: (i, 0)),
            pl.BlockSpec((rblk, _BEAM), lambda i: (i, 0)),
        ],
        out_shape=[
            jax.ShapeDtypeStruct((n, 1), jnp.float32),
            jax.ShapeDtypeStruct((n, 1), jnp.float32),
            jax.ShapeDtypeStruct((n, _BEAM), jnp.float32),
            jax.ShapeDtypeStruct((n, _BEAM), jnp.int32),
        ],
    )(out)


def _merge_body(v, m_ref, s_ref, sc_ref, tv_ref, ti_ref, ns_ref, x_ref, ptr_ref):
    m = m_ref[...]
    s = s_ref[...]
    sc = sc_ref[...]
    tv = tv_ref[...]
    ti = ti_ref[...]
    b, c = tv.shape
    cand = jnp.exp(tv - m) / s + sc
    iota = lax.broadcasted_iota(jnp.int32, (b, c), 1)
    flat = (iota // _BEAM) * v + ti
    big = jnp.int32(_BEAM * v)
    for k in range(_BEAM):
        vk = jnp.max(cand, axis=1, keepdims=True)
        fk = jnp.min(jnp.where(cand == vk, flat, big), axis=1, keepdims=True)
        ns_ref[:, k : k + 1] = jnp.where(fk % v == _EOS, _NEG_INF, vk)
        x_ref[:, k : k + 1] = fk % v
        ptr_ref[:, k : k + 1] = fk // v
        cand = jnp.where((cand == vk) & (flat == fk), _NEG_INF, cand)


def _merge(m64, s64, sc64, tv64, ti64, v):
    b, c = tv64.shape
    return pl.pallas_call(
        functools.partial(_merge_body, v),
        out_shape=[
            jax.ShapeDtypeStruct((b, _BEAM), jnp.float32),
            jax.ShapeDtypeStruct((b, _BEAM), jnp.int32),
            jax.ShapeDtypeStruct((b, _BEAM), jnp.int32),
        ],
    )(m64, s64, sc64, tv64, ti64)


def _sc_gather(hf, idx):
    rows, d = hf.shape
    bpw = rows // _NW
    mesh = plsc.VectorSubcoreMesh(core_axis_name="c", subcore_axis_name="s")

    @functools.partial(
        pl.kernel,
        mesh=mesh,
        out_type=jax.ShapeDtypeStruct((rows, d), jnp.float32),
        scratch_types=[
            pltpu.VMEM((bpw,), jnp.int32),
            pltpu.VMEM((bpw, d), jnp.float32),
            pltpu.SemaphoreType.DMA,
        ],
    )
    def gather_k(hf_hbm, idx_hbm, out_hbm, idx_v, rows_v, sem):
        wid = lax.axis_index("s") * _NC + lax.axis_index("c")
        base = wid * bpw
        pltpu.sync_copy(idx_hbm.at[pl.ds(base, bpw)], idx_v)
        pltpu.async_copy(hf_hbm.at[idx_v], rows_v, sem).wait()
        pltpu.sync_copy(rows_v, out_hbm.at[pl.ds(base, bpw)])

    return gather_k(hf, idx)


def kernel(out, score, h, beam_size):
    n, v = out.shape
    b = n // _BEAM
    l, _, hd = h.shape

    m, s, topv, topi = _stats_topk(out)

    m64 = jnp.repeat(m.reshape(b, _BEAM), _BEAM, axis=1)
    s64 = jnp.repeat(s.reshape(b, _BEAM), _BEAM, axis=1)
    sc64 = jnp.repeat(score.reshape(b, _BEAM), _BEAM, axis=1)
    tv64 = topv.reshape(b, _BEAM * _BEAM)
    ti64 = topi.reshape(b, _BEAM * _BEAM)

    new_score, xt, beam = _merge(m64, s64, sc64, tv64, ti64, v)

    batch_base = jnp.arange(b, dtype=jnp.int32) * _BEAM
    ptr = jnp.reshape(beam + batch_base[:, None], (-1,))
    x = jnp.reshape(xt, (-1,))

    hf = h.reshape(l * n, hd)
    layer_off = jnp.repeat(jnp.arange(l, dtype=jnp.int32) * n, n)
    flat_idx = jnp.tile(ptr, l) + layer_off
    h_new = _sc_gather(hf, flat_idx).reshape(l, n, hd)

    return new_score, x, ptr, h_new


# speedup vs baseline: 1.5722x; 1.0510x over previous
import functools

import jax
import jax.numpy as jnp
import numpy as np
from jax import lax
from jax.experimental import pallas as pl
from jax.experimental.pallas import tpu as pltpu
from jax.experimental.pallas import tpu_sc as plsc

_EOS = 2
_BEAM = 8
_NEG_INF = np.float32(-np.inf)

_NC = 2
_NS = 16
_NW = _NC * _NS


_CW = 128
_DEPTH = 3


def _ce(v, jv, regs, jregs):
    for d in range(_DEPTH - 1):
        c = v > regs[d]
        regs[d], v = jnp.where(c, v, regs[d]), jnp.where(c, regs[d], v)
        jregs[d], jv = jnp.where(c, jv, jregs[d]), jnp.where(c, jregs[d], jv)
    c = v > regs[_DEPTH - 1]
    regs[_DEPTH - 1] = jnp.where(c, v, regs[_DEPTH - 1])
    jregs[_DEPTH - 1] = jnp.where(c, jv, jregs[_DEPTH - 1])
    return regs, jregs


def _stats_topk_body(out_ref, m_ref, s_ref, topv_ref, topi_ref):
    rb = out_ref.shape[0]
    v = out_ref.shape[1]
    nfull = v // _CW
    rem = v - nfull * _CW
    lane = lax.broadcasted_iota(jnp.int32, (rb, _CW), 1)

    ninf = jnp.full((rb, _CW), _NEG_INF)
    zi = jnp.zeros((rb, _CW), jnp.int32)
    init = (ninf, ninf, ninf, zi, zi, zi)

    def s1(j, carry):
        regs, jregs = list(carry[:_DEPTH]), list(carry[_DEPTH:])
        base = j * _CW
        blk = out_ref[:, pl.ds(base, _CW)]
        jb = jnp.full((rb, _CW), base, jnp.int32)
        regs, jregs = _ce(blk, jb, regs, jregs)
        return (*regs, *jregs)

    carry = lax.fori_loop(0, nfull, s1, init, unroll=4)
    regs, jregs = list(carry[:_DEPTH]), list(carry[_DEPTH:])
    if rem:
        blk = out_ref[:, v - _CW : v]
        blk = jnp.where(lane < _CW - rem, _NEG_INF, blk)
        jb = jnp.full((rb, _CW), v - _CW, jnp.int32)
        regs, jregs = _ce(blk, jb, regs, jregs)
    r0 = regs[0]

    m = jnp.max(r0, axis=1, keepdims=True)
    m_ref[...] = m

    rc = r0
    for _ in range(_BEAM - 1):
        vk = jnp.max(rc, axis=1, keepdims=True)
        rc = jnp.where(rc == vk, _NEG_INF, rc)
    t = jnp.max(rc, axis=1, keepdims=True)

    def s2(j, carry):
        acc, cnt = carry
        blk = out_ref[:, pl.ds(j * _CW, _CW)]
        acc = acc + jnp.exp(blk - m)
        cnt = cnt + (blk >= t).astype(jnp.int32)
        return acc, cnt

    acc, cnt = lax.fori_loop(0, nfull, s2, (jnp.zeros((rb, _CW)), zi), unroll=4)
    if rem:
        blk = out_ref[:, v - _CW : v]
        blk = jnp.where(lane < _CW - rem, _NEG_INF, blk)
        acc = acc + jnp.exp(blk - m)
        cnt = cnt + (blk >= t).astype(jnp.int32)
    s_ref[...] = jnp.sum(acc, axis=1, keepdims=True)

    bad = jnp.max(cnt) > _DEPTH

    @pl.when(jnp.logical_not(bad))
    def _fast():
        cv = jnp.concatenate(regs, axis=1)
        li = lax.broadcasted_iota(jnp.int32, (rb, _DEPTH * _CW), 1) & (
            _CW - 1
        )
        ci = jnp.concatenate(jregs, axis=1) + li
        big = jnp.int32(v)
        for k in range(_BEAM):
            vk = jnp.max(cv, axis=1, keepdims=True)
            fi = jnp.min(jnp.where(cv == vk, ci, big), axis=1, keepdims=True)
            topv_ref[:, k : k + 1] = vk
            topi_ref[:, k : k + 1] = fi
            cv = jnp.where((cv == vk) & (ci == fi), _NEG_INF, cv)

    @pl.when(bad)
    def _slow():
        x = out_ref[...]
        iota = lax.broadcasted_iota(jnp.int32, (rb, v), 1)
        big = jnp.int32(v)
        for k in range(_BEAM):
            vk = jnp.max(x, axis=1, keepdims=True)
            fi = jnp.min(jnp.where(x == vk, iota, big), axis=1, keepdims=True)
            topv_ref[:, k : k + 1] = vk
            topi_ref[:, k : k + 1] = fi
            x = jnp.where(iota == fi, _NEG_INF, x)


def _stats_topk(out):
    n, v = out.shape
    rblk = 32
    grid = n // rblk
    return pl.pallas_call(
        _stats_topk_body,
        grid=(grid,),
        in_specs=[pl.BlockSpec((rblk, v), lambda i: (i, 0))],
        out_specs=[
            pl.BlockSpec((rblk, 1), lambda i: (i, 0)),
            pl.BlockSpec((rblk, 1), lambda i: (i, 0)),
            pl.BlockSpec((rblk, _BEAM), lambda i: (i, 0)),
            pl.BlockSpec((rblk, _BEAM), lambda i: (i, 0)),
        ],
        out_shape=[
            jax.ShapeDtypeStruct((n, 1), jnp.float32),
            jax.ShapeDtypeStruct((n, 1), jnp.float32),
            jax.ShapeDtypeStruct((n, _BEAM), jnp.float32),
            jax.ShapeDtypeStruct((n, _BEAM), jnp.int32),
        ],
    )(out)


def _merge_body(v, m_ref, s_ref, sc_ref, tv_ref, ti_ref, ns_ref, x_ref, ptr_ref):
    m = m_ref[...]
    s = s_ref[...]
    sc = sc_ref[...]
    tv = tv_ref[...]
    ti = ti_ref[...]
    b, c = tv.shape
    cand = jnp.exp(tv - m) / s + sc
    iota = lax.broadcasted_iota(jnp.int32, (b, c), 1)
    flat = (iota // _BEAM) * v + ti
    big = jnp.int32(_BEAM * v)
    for k in range(_BEAM):
        vk = jnp.max(cand, axis=1, keepdims=True)
        fk = jnp.min(jnp.where(cand == vk, flat, big), axis=1, keepdims=True)
        ns_ref[:, k : k + 1] = jnp.where(fk % v == _EOS, _NEG_INF, vk)
        x_ref[:, k : k + 1] = fk % v
        ptr_ref[:, k : k + 1] = fk // v
        cand = jnp.where((cand == vk) & (flat == fk), _NEG_INF, cand)


def _merge(m64, s64, sc64, tv64, ti64, v):
    b, c = tv64.shape
    return pl.pallas_call(
        functools.partial(_merge_body, v),
        out_shape=[
            jax.ShapeDtypeStruct((b, _BEAM), jnp.float32),
            jax.ShapeDtypeStruct((b, _BEAM), jnp.int32),
            jax.ShapeDtypeStruct((b, _BEAM), jnp.int32),
        ],
    )(m64, s64, sc64, tv64, ti64)


def _sc_gather(hf, idx):
    rows, d = hf.shape
    bpw = rows // _NW
    mesh = plsc.VectorSubcoreMesh(core_axis_name="c", subcore_axis_name="s")

    @functools.partial(
        pl.kernel,
        mesh=mesh,
        out_type=jax.ShapeDtypeStruct((rows, d), jnp.float32),
        scratch_types=[
            pltpu.VMEM((bpw,), jnp.int32),
            pltpu.VMEM((bpw, d), jnp.float32),
            pltpu.SemaphoreType.DMA,
        ],
    )
    def gather_k(hf_hbm, idx_hbm, out_hbm, idx_v, rows_v, sem):
        wid = lax.axis_index("s") * _NC + lax.axis_index("c")
        base = wid * bpw
        pltpu.sync_copy(idx_hbm.at[pl.ds(base, bpw)], idx_v)
        pltpu.async_copy(hf_hbm.at[idx_v], rows_v, sem).wait()
        pltpu.sync_copy(rows_v, out_hbm.at[pl.ds(base, bpw)])

    return gather_k(hf, idx)


def kernel(out, score, h, beam_size):
    n, v = out.shape
    b = n // _BEAM
    l, _, hd = h.shape

    m, s, topv, topi = _stats_topk(out)

    m64 = jnp.repeat(m.reshape(b, _BEAM), _BEAM, axis=1)
    s64 = jnp.repeat(s.reshape(b, _BEAM), _BEAM, axis=1)
    sc64 = jnp.repeat(score.reshape(b, _BEAM), _BEAM, axis=1)
    tv64 = topv.reshape(b, _BEAM * _BEAM)
    ti64 = topi.reshape(b, _BEAM * _BEAM)

    new_score, xt, beam = _merge(m64, s64, sc64, tv64, ti64, v)

    batch_base = jnp.arange(b, dtype=jnp.int32) * _BEAM
    ptr = jnp.reshape(beam + batch_base[:, None], (-1,))
    x = jnp.reshape(xt, (-1,))

    hf = h.reshape(l * n, hd)
    layer_off = jnp.repeat(jnp.arange(l, dtype=jnp.int32) * n, n)
    flat_idx = jnp.tile(ptr, l) + layer_off
    h_new = _sc_gather(hf, flat_idx).reshape(l, n, hd)

    return new_score, x, ptr, h_new


# speedup vs baseline: 1.6762x; 1.0661x over previous
import functools

import jax
import jax.numpy as jnp
import numpy as np
from jax import lax
from jax.experimental import pallas as pl
from jax.experimental.pallas import tpu as pltpu
from jax.experimental.pallas import tpu_sc as plsc

_EOS = 2
_BEAM = 8
_NEG_INF = np.float32(-np.inf)

_NC = 2
_NS = 16
_NW = _NC * _NS


_CW = 128
_DEPTH = 3


def _ce(v, jv, regs, jregs):
    for d in range(_DEPTH - 1):
        c = v > regs[d]
        regs[d], v = jnp.where(c, v, regs[d]), jnp.where(c, regs[d], v)
        jregs[d], jv = jnp.where(c, jv, jregs[d]), jnp.where(c, jregs[d], jv)
    c = v > regs[_DEPTH - 1]
    regs[_DEPTH - 1] = jnp.where(c, v, regs[_DEPTH - 1])
    jregs[_DEPTH - 1] = jnp.where(c, jv, jregs[_DEPTH - 1])
    return regs, jregs


def _stats_topk_body(out_ref, m_ref, s_ref, topv_ref, topi_ref):
    rb = out_ref.shape[0]
    v = out_ref.shape[1]
    nfull = v // _CW
    rem = v - nfull * _CW
    lane = lax.broadcasted_iota(jnp.int32, (rb, _CW), 1)

    ninf = jnp.full((rb, _CW), _NEG_INF)
    zi = jnp.zeros((rb, _CW), jnp.int32)
    init = (ninf, ninf, ninf, zi, zi, zi)

    def s1(j, carry):
        regs, jregs = list(carry[:_DEPTH]), list(carry[_DEPTH:])
        base = j * _CW
        blk = out_ref[:, pl.ds(base, _CW)]
        jb = jnp.full((rb, _CW), base, jnp.int32)
        regs, jregs = _ce(blk, jb, regs, jregs)
        return (*regs, *jregs)

    carry = lax.fori_loop(0, nfull, s1, init, unroll=8)
    regs, jregs = list(carry[:_DEPTH]), list(carry[_DEPTH:])
    if rem:
        blk = out_ref[:, v - _CW : v]
        blk = jnp.where(lane < _CW - rem, _NEG_INF, blk)
        jb = jnp.full((rb, _CW), v - _CW, jnp.int32)
        regs, jregs = _ce(blk, jb, regs, jregs)
    r0 = regs[0]

    m = jnp.max(r0, axis=1, keepdims=True)
    m_ref[...] = m

    rc = r0
    for _ in range(_BEAM - 1):
        vk = jnp.max(rc, axis=1, keepdims=True)
        rc = jnp.where(rc == vk, _NEG_INF, rc)
    t = jnp.max(rc, axis=1, keepdims=True)

    def s2(j, carry):
        acc, cnt = carry
        blk = out_ref[:, pl.ds(j * _CW, _CW)]
        acc = acc + jnp.exp(blk - m)
        cnt = cnt + (blk >= t).astype(jnp.int32)
        return acc, cnt

    acc, cnt = lax.fori_loop(0, nfull, s2, (jnp.zeros((rb, _CW)), zi), unroll=8)
    if rem:
        blk = out_ref[:, v - _CW : v]
        blk = jnp.where(lane < _CW - rem, _NEG_INF, blk)
        acc = acc + jnp.exp(blk - m)
        cnt = cnt + (blk >= t).astype(jnp.int32)
    s_ref[...] = jnp.sum(acc, axis=1, keepdims=True)

    bad = jnp.max(cnt) > _DEPTH

    @pl.when(jnp.logical_not(bad))
    def _fast():
        cv = jnp.concatenate(regs, axis=1)
        li = lax.broadcasted_iota(jnp.int32, (rb, _DEPTH * _CW), 1) & (
            _CW - 1
        )
        ci = jnp.concatenate(jregs, axis=1) + li
        big = jnp.int32(v)
        for k in range(_BEAM):
            vk = jnp.max(cv, axis=1, keepdims=True)
            fi = jnp.min(jnp.where(cv == vk, ci, big), axis=1, keepdims=True)
            topv_ref[:, k : k + 1] = vk
            topi_ref[:, k : k + 1] = fi
            cv = jnp.where((cv == vk) & (ci == fi), _NEG_INF, cv)

    @pl.when(bad)
    def _slow():
        x = out_ref[...]
        iota = lax.broadcasted_iota(jnp.int32, (rb, v), 1)
        big = jnp.int32(v)
        for k in range(_BEAM):
            vk = jnp.max(x, axis=1, keepdims=True)
            fi = jnp.min(jnp.where(x == vk, iota, big), axis=1, keepdims=True)
            topv_ref[:, k : k + 1] = vk
            topi_ref[:, k : k + 1] = fi
            x = jnp.where(iota == fi, _NEG_INF, x)


def _stats_topk(out):
    n, v = out.shape
    rblk = 32
    grid = n // rblk
    return pl.pallas_call(
        _stats_topk_body,
        grid=(grid,),
        in_specs=[pl.BlockSpec((rblk, v), lambda i: (i, 0))],
        out_specs=[
            pl.BlockSpec((rblk, 1), lambda i: (i, 0)),
            pl.BlockSpec((rblk, 1), lambda i: (i, 0)),
            pl.BlockSpec((rblk, _BEAM), lambda i: (i, 0)),
            pl.BlockSpec((rblk, _BEAM), lambda i: (i, 0)),
        ],
        out_shape=[
            jax.ShapeDtypeStruct((n, 1), jnp.float32),
            jax.ShapeDtypeStruct((n, 1), jnp.float32),
            jax.ShapeDtypeStruct((n, _BEAM), jnp.float32),
            jax.ShapeDtypeStruct((n, _BEAM), jnp.int32),
        ],
    )(out)


def _merge_body(v, m_ref, s_ref, sc_ref, tv_ref, ti_ref, ns_ref, x_ref, ptr_ref):
    m = m_ref[...]
    s = s_ref[...]
    sc = sc_ref[...]
    tv = tv_ref[...]
    ti = ti_ref[...]
    b, c = tv.shape
    cand = jnp.exp(tv - m) / s + sc
    iota = lax.broadcasted_iota(jnp.int32, (b, c), 1)
    flat = (iota // _BEAM) * v + ti
    big = jnp.int32(_BEAM * v)
    for k in range(_BEAM):
        vk = jnp.max(cand, axis=1, keepdims=True)
        fk = jnp.min(jnp.where(cand == vk, flat, big), axis=1, keepdims=True)
        ns_ref[:, k : k + 1] = jnp.where(fk % v == _EOS, _NEG_INF, vk)
        x_ref[:, k : k + 1] = fk % v
        ptr_ref[:, k : k + 1] = fk // v
        cand = jnp.where((cand == vk) & (flat == fk), _NEG_INF, cand)


def _merge(m64, s64, sc64, tv64, ti64, v):
    b, c = tv64.shape
    return pl.pallas_call(
        functools.partial(_merge_body, v),
        out_shape=[
            jax.ShapeDtypeStruct((b, _BEAM), jnp.float32),
            jax.ShapeDtypeStruct((b, _BEAM), jnp.int32),
            jax.ShapeDtypeStruct((b, _BEAM), jnp.int32),
        ],
    )(m64, s64, sc64, tv64, ti64)


def _sc_gather(hf, idx):
    rows, d = hf.shape
    bpw = rows // _NW
    mesh = plsc.VectorSubcoreMesh(core_axis_name="c", subcore_axis_name="s")

    @functools.partial(
        pl.kernel,
        mesh=mesh,
        out_type=jax.ShapeDtypeStruct((rows, d), jnp.float32),
        scratch_types=[
            pltpu.VMEM((bpw,), jnp.int32),
            pltpu.VMEM((bpw, d), jnp.float32),
            pltpu.SemaphoreType.DMA,
        ],
    )
    def gather_k(hf_hbm, idx_hbm, out_hbm, idx_v, rows_v, sem):
        wid = lax.axis_index("s") * _NC + lax.axis_index("c")
        base = wid * bpw
        pltpu.sync_copy(idx_hbm.at[pl.ds(base, bpw)], idx_v)
        pltpu.async_copy(hf_hbm.at[idx_v], rows_v, sem).wait()
        pltpu.sync_copy(rows_v, out_hbm.at[pl.ds(base, bpw)])

    return gather_k(hf, idx)


def kernel(out, score, h, beam_size):
    n, v = out.shape
    b = n // _BEAM
    l, _, hd = h.shape

    m, s, topv, topi = _stats_topk(out)

    m64 = jnp.repeat(m.reshape(b, _BEAM), _BEAM, axis=1)
    s64 = jnp.repeat(s.reshape(b, _BEAM), _BEAM, axis=1)
    sc64 = jnp.repeat(score.reshape(b, _BEAM), _BEAM, axis=1)
    tv64 = topv.reshape(b, _BEAM * _BEAM)
    ti64 = topi.reshape(b, _BEAM * _BEAM)

    new_score, xt, beam = _merge(m64, s64, sc64, tv64, ti64, v)

    batch_base = jnp.arange(b, dtype=jnp.int32) * _BEAM
    ptr = jnp.reshape(beam + batch_base[:, None], (-1,))
    x = jnp.reshape(xt, (-1,))

    hf = h.reshape(l * n, hd)
    layer_off = jnp.repeat(jnp.arange(l, dtype=jnp.int32) * n, n)
    flat_idx = jnp.tile(ptr, l) + layer_off
    h_new = _sc_gather(hf, flat_idx).reshape(l, n, hd)

    return new_score, x, ptr, h_new
